# baseline (speedup 1.0000x reference)
import functools
from math import pi

import jax
import jax.numpy as jnp
from jax import lax
from jax.experimental import pallas as pl
from jax.experimental.pallas import tpu as pltpu
from jax.experimental.pallas import tpu_sc as plsc

N_PAD = 10240
KNN = 32
FILT = 128
HID = 256
NRAD = 32
CUTOFF = 5.0
CUT2 = CUTOFF * CUTOFF
BIG = 1e10
BIGI = 2 ** 30

R = 1024
NBR = N_PAD // R
CT = 512
WW = KNN + CT
E = N_PAD * KNN

EB = 2048
RB = 1024


def _gelu(v):
    return jnp.float32(0.5) * v * (jnp.float32(1.0) + lax.erf(v * jnp.float32(0.7071067811865476)))


def _knn_body(tlo_ref, thi_ref, posr_ref, poscT_ref, br_ref, bc_ref,
              d2n_out, d2p_out, idx_out,
              wd_ref, wp_ref, wi_ref, bd_ref, bp_ref, bi_ref):
    i = pl.program_id(0)
    r0 = i * R
    bd_ref[...] = jnp.full((R, KNN), BIG, jnp.float32)
    bp_ref[...] = jnp.full((R, KNN), BIG, jnp.float32)
    bi_ref[...] = jnp.zeros((R, KNN), jnp.int32)
    posr = posr_ref[...]
    posr_b = posr.astype(jnp.bfloat16)
    sqr = posr[:, 0:1] * posr[:, 0:1] + posr[:, 1:2] * posr[:, 1:2] + posr[:, 2:3] * posr[:, 2:3]
    br = br_ref[...]
    rowid = r0 + lax.broadcasted_iota(jnp.int32, (R, CT), 0)
    lane_w = lax.broadcasted_iota(jnp.int32, (R, WW), 1)
    lane_k = lax.broadcasted_iota(jnp.int32, (R, KNN), 1)
    tlo = tlo_ref[i]
    thi = thi_ref[i]

    def tile_body(t, carry):
        c0 = t * CT
        pc = poscT_ref[:, pl.ds(c0, CT)]
        pc_b = pc.astype(jnp.bfloat16)
        d0 = posr[:, 0:1] - pc[0:1, :]
        d2p = d0 * d0
        d1 = posr[:, 1:2] - pc[1:2, :]
        d2p = d2p + d1 * d1
        dz = posr[:, 2:3] - pc[2:3, :]
        d2p = d2p + dz * dz
        sqc = pc[0:1, :] * pc[0:1, :] + pc[1:2, :] * pc[1:2, :] + pc[2:3, :] * pc[2:3, :]
        dot = jnp.dot(posr_b, pc_b, preferred_element_type=jnp.float32)
        d2n = jnp.maximum(sqr + sqc - 2.0 * dot, 0.0)
        bc = bc_ref[:, pl.ds(c0, CT)]
        colid = c0 + lax.broadcasted_iota(jnp.int32, (R, CT), 1)
        valid = (bc == br) & (colid != rowid)
        d2m = jnp.where(valid, d2n, BIG)
        rowmin = jnp.min(d2m, axis=1, keepdims=True)
        bestmax = jnp.max(bd_ref[...], axis=1, keepdims=True)

        @pl.when(jnp.any(rowmin < bestmax))
        def _():
            wd_ref[:, 0:KNN] = bd_ref[...]
            wd_ref[:, KNN:WW] = d2m
            wp_ref[:, 0:KNN] = bp_ref[...]
            wp_ref[:, KNN:WW] = d2p
            wi_ref[...] = bi_ref[...]

            def sel(s, c2):
                wd = wd_ref[...]
                m = jnp.min(wd, axis=1, keepdims=True)
                am = jnp.min(jnp.where(wd == m, lane_w, BIGI), axis=1, keepdims=True)
                hit = lane_w == am
                iv_old = jnp.sum(jnp.where(lane_k == am, wi_ref[...], 0), axis=1, keepdims=True)
                iv = jnp.where(am < KNN, iv_old, am - KNN + c0)
                pv = jnp.sum(jnp.where(hit, wp_ref[...], 0.0), axis=1, keepdims=True)
                bd_ref[...] = jnp.where(lane_k == s, m, bd_ref[...])
                bp_ref[...] = jnp.where(lane_k == s, pv, bp_ref[...])
                bi_ref[...] = jnp.where(lane_k == s, iv, bi_ref[...])
                wd_ref[...] = jnp.where(hit, BIG, wd)
                return c2

            lax.fori_loop(0, KNN, sel, 0)
        return carry

    lax.fori_loop(tlo, thi, tile_body, 0)
    d2n_out[...] = bd_ref[...]
    d2p_out[...] = bp_ref[...]
    idx_out[...] = bi_ref[...]


def _knn_call(pos8, poscT, br, bc, tlo, thi):
    return pl.pallas_call(
        _knn_body,
        grid=(NBR,),
        in_specs=[
            pl.BlockSpec(memory_space=pltpu.SMEM),
            pl.BlockSpec(memory_space=pltpu.SMEM),
            pl.BlockSpec((R, 8), lambda i: (i, 0)),
            pl.BlockSpec((8, N_PAD), lambda i: (0, 0)),
            pl.BlockSpec((R, 1), lambda i: (i, 0)),
            pl.BlockSpec((1, N_PAD), lambda i: (0, 0)),
        ],
        out_specs=[
            pl.BlockSpec((R, KNN), lambda i: (i, 0)),
            pl.BlockSpec((R, KNN), lambda i: (i, 0)),
            pl.BlockSpec((R, KNN), lambda i: (i, 0)),
        ],
        out_shape=[
            jax.ShapeDtypeStruct((N_PAD, KNN), jnp.float32),
            jax.ShapeDtypeStruct((N_PAD, KNN), jnp.float32),
            jax.ShapeDtypeStruct((N_PAD, KNN), jnp.int32),
        ],
        scratch_shapes=[
            pltpu.VMEM((R, WW), jnp.float32),
            pltpu.VMEM((R, WW), jnp.float32),
            pltpu.VMEM((R, KNN), jnp.int32),
            pltpu.VMEM((R, KNN), jnp.float32),
            pltpu.VMEM((R, KNN), jnp.float32),
            pltpu.VMEM((R, KNN), jnp.int32),
        ],
    )(tlo, thi, pos8, poscT, br, bc)


def _edge_body(d2n_ref, d2p_ref, wr_ref, o0, rbf_ref):
    mask = d2n_ref[0] <= jnp.float32(CUT2)
    dist = jnp.where(mask, jnp.sqrt(d2p_ref[0] + jnp.float32(1e-12)), jnp.float32(0.5 * CUTOFF))
    xq = dist * jnp.float32(1.0 / CUTOFF)
    x2 = xq * xq
    x4 = x2 * x2
    x5 = x4 * xq
    x6 = x4 * x2
    x7 = x6 * xq
    env = jnp.float32(1.0) / xq + jnp.float32(-28.0) * x5 + jnp.float32(48.0) * x6 + jnp.float32(-21.0) * x7
    theta = xq * jnp.float32(pi)
    s1 = jnp.sin(theta)
    c2 = jnp.float32(2.0) * jnp.cos(theta)
    cosw = jnp.float32(0.5) * (jnp.cos(dist * jnp.float32(pi / CUTOFF)) + jnp.float32(1.0))
    cm = cosw * mask.astype(jnp.float32)
    u = env * cm
    sm1 = jnp.zeros_like(s1)
    sb = s1
    for b in range(NRAD):
        rbf_ref[b:b + 1, :] = u * sb
        snext = c2 * sb - sm1
        sm1 = sb
        sb = snext
    rbf_ref[NRAD:NRAD + 1, :] = cm
    rbf_ref[NRAD + 1:40, :] = jnp.zeros((40 - NRAD - 1, EB), jnp.float32)
    basis = rbf_ref[...]
    dn = (((0,), (0,)), ((), ()))
    o0[...] = lax.dot_general(basis, wr_ref[...], dn, preferred_element_type=jnp.float32)


def _edge_call(d2nf, d2pf, wr_aug_b):
    return pl.pallas_call(
        _edge_body,
        grid=(E // EB,),
        in_specs=[
            pl.BlockSpec((1, 1, EB), lambda e: (e, 0, 0)),
            pl.BlockSpec((1, 1, EB), lambda e: (e, 0, 0)),
            pl.BlockSpec((40, FILT), lambda e: (0, 0)),
        ],
        out_specs=pl.BlockSpec((EB, FILT), lambda e: (e, 0)),
        out_shape=jax.ShapeDtypeStruct((E, FILT), jnp.float32),
        scratch_shapes=[pltpu.VMEM((40, EB), jnp.float32)],
    )(d2nf, d2pf, wr_aug_b)


def _emb_body(z_ref, et_ref, we_ref, be_ref, wl1_ref, x_out, h_out):
    z = z_ref[...]
    oh = (z == lax.broadcasted_iota(jnp.int32, (RB, 128), 1)).astype(jnp.float32)
    t = jnp.dot(oh, et_ref[...], preferred_element_type=jnp.float32)
    v = jnp.dot(t, we_ref[...], preferred_element_type=jnp.float32) + be_ref[...]
    x = _gelu(v)
    x_out[...] = x
    h_out[...] = jnp.dot(x, wl1_ref[...], preferred_element_type=jnp.float32)


def _emb_call(zp, etP, weP, be, wl1):
    return pl.pallas_call(
        _emb_body,
        grid=(N_PAD // RB,),
        in_specs=[
            pl.BlockSpec((RB, 1), lambda i: (i, 0)),
            pl.BlockSpec((128, 128), lambda i: (0, 0)),
            pl.BlockSpec((128, HID), lambda i: (0, 0)),
            pl.BlockSpec((1, HID), lambda i: (0, 0)),
            pl.BlockSpec((HID, FILT), lambda i: (0, 0)),
        ],
        out_specs=[
            pl.BlockSpec((RB, HID), lambda i: (i, 0)),
            pl.BlockSpec((RB, FILT), lambda i: (i, 0)),
        ],
        out_shape=[
            jax.ShapeDtypeStruct((N_PAD, HID), jnp.float32),
            jax.ShapeDtypeStruct((N_PAD, FILT), jnp.float32),
        ],
    )(zp, etP, weP, be, wl1)


def _blk_body(x_ref, agg_ref, wl2_ref, bl2_ref, wf1_ref, bf1_ref, wf2_ref, bf2_ref,
              wn_ref, bn_ref, x_out, n_out):
    x1 = x_ref[...] + jnp.dot(agg_ref[...], wl2_ref[...], preferred_element_type=jnp.float32) + bl2_ref[...]
    t = _gelu(jnp.dot(x1, wf1_ref[...], preferred_element_type=jnp.float32) + bf1_ref[...])
    x2 = x1 + jnp.dot(t, wf2_ref[...], preferred_element_type=jnp.float32) + bf2_ref[...]
    x_out[...] = x2
    n_out[...] = jnp.dot(x2, wn_ref[...], preferred_element_type=jnp.float32) + bn_ref[...]


def _blk_call(x, agg, wl2, bl2, wf1, bf1, wf2, bf2, wn, bn):
    nout = wn.shape[1]
    return pl.pallas_call(
        _blk_body,
        grid=(N_PAD // RB,),
        in_specs=[
            pl.BlockSpec((RB, HID), lambda i: (i, 0)),
            pl.BlockSpec((RB, FILT), lambda i: (i, 0)),
            pl.BlockSpec((FILT, HID), lambda i: (0, 0)),
            pl.BlockSpec((1, HID), lambda i: (0, 0)),
            pl.BlockSpec((HID, HID), lambda i: (0, 0)),
            pl.BlockSpec((1, HID), lambda i: (0, 0)),
            pl.BlockSpec((HID, HID), lambda i: (0, 0)),
            pl.BlockSpec((1, HID), lambda i: (0, 0)),
            pl.BlockSpec((HID, nout), lambda i: (0, 0)),
            pl.BlockSpec((1, nout), lambda i: (0, 0)),
        ],
        out_specs=[
            pl.BlockSpec((RB, HID), lambda i: (i, 0)),
            pl.BlockSpec((RB, nout), lambda i: (i, 0)),
        ],
        out_shape=[
            jax.ShapeDtypeStruct((N_PAD, HID), jnp.float32),
            jax.ShapeDtypeStruct((N_PAD, nout), jnp.float32),
        ],
    )(x, agg, wl2, bl2, wf1, bf1, wf2, bf2, wn, bn)


NW = 32
NPT = N_PAD // NW
G = 2
GI = G * KNN
NG = NPT // G


def _msg_sc_body(h_hbm, wm_hbm, src_hbm, out_hbm, idx_v, rows0, rows1, wmv0, wmv1,
                 agg_v, gs0, gs1, ws0, ws1):
    cid = lax.axis_index("c")
    sid = lax.axis_index("s")
    wid = sid * 2 + cid
    base = wid * NPT
    pltpu.sync_copy(src_hbm.at[pl.ds(base * KNN, NPT * KNN)], idx_v)

    rows_b = (rows0, rows1)
    wm_b = (wmv0, wmv1)
    gs_b = (gs0, gs1)
    ws_b = (ws0, ws1)

    def issue(g, buf):
        pltpu.async_copy(h_hbm.at[idx_v.at[pl.ds(g * GI, GI)]], rows_b[buf], gs_b[buf])
        pltpu.async_copy(wm_hbm.at[pl.ds(base * KNN + g * GI, GI)], wm_b[buf], ws_b[buf])

    issue(0, 0)

    def body(g2, carry):
        for par in (0, 1):
            g = 2 * g2 + par
            cur = par
            nxt = 1 - par

            @pl.when(g + 1 < NG)
            def _():
                pltpu.async_copy(h_hbm.at[idx_v.at[pl.ds((g + 1) * GI, GI)]], rows_b[nxt], gs_b[nxt])
                pltpu.async_copy(wm_hbm.at[pl.ds(base * KNN + (g + 1) * GI, GI)], wm_b[nxt], ws_b[nxt])

            pltpu.make_async_copy(h_hbm.at[idx_v.at[pl.ds(0, GI)]], rows_b[cur], gs_b[cur]).wait()
            pltpu.make_async_copy(wm_hbm.at[pl.ds(0, GI)], wm_b[cur], ws_b[cur]).wait()

            for gn in range(G):
                n = g * G + gn
                for c in range(FILT // 16):
                    acc = jnp.zeros((16,), jnp.float32)
                    for s in range(KNN):
                        e = gn * KNN + s
                        acc = acc + rows_b[cur][e, pl.ds(c * 16, 16)] * wm_b[cur][e, pl.ds(c * 16, 16)]
                    agg_v[n, pl.ds(c * 16, 16)] = acc
        return carry

    lax.fori_loop(0, NG // 2, body, 0)
    pltpu.sync_copy(agg_v, out_hbm.at[pl.ds(base, NPT)])


def _msg_call(h, wm, idxf):
    mesh = plsc.VectorSubcoreMesh(core_axis_name="c", subcore_axis_name="s")
    fn = functools.partial(
        pl.kernel,
        mesh=mesh,
        out_type=jax.ShapeDtypeStruct((N_PAD, FILT), jnp.float32),
        scratch_types=[
            pltpu.VMEM((NPT * KNN,), jnp.int32),
            pltpu.VMEM((GI, FILT), jnp.float32),
            pltpu.VMEM((GI, FILT), jnp.float32),
            pltpu.VMEM((GI, FILT), jnp.float32),
            pltpu.VMEM((GI, FILT), jnp.float32),
            pltpu.VMEM((NPT, FILT), jnp.float32),
            pltpu.SemaphoreType.DMA,
            pltpu.SemaphoreType.DMA,
            pltpu.SemaphoreType.DMA,
            pltpu.SemaphoreType.DMA,
        ],
    )(_msg_sc_body)
    return fn(h, wm, idxf)


def kernel(z, pos, batch, emb_table, W_emb, b_emb, freq, W_rbf, b_rbf,
           W_lin1, W_lin2, b_lin2, W_fc1, b_fc1, W_fc2, b_fc2, W_out, b_out):
    n = pos.shape[0]
    npad = N_PAD - n

    posf = pos.astype(jnp.float32)
    pos8 = jnp.concatenate(
        [posf, jnp.zeros((n, 5), jnp.float32)], axis=1)
    pos8 = jnp.concatenate([pos8, jnp.zeros((npad, 8), jnp.float32)], axis=0)
    poscT = pos8.T

    bi = batch.astype(jnp.int32)
    br = jnp.concatenate([bi, jnp.full((npad,), bi[-1], jnp.int32)]).reshape(N_PAD, 1)
    bc = jnp.concatenate([bi, jnp.full((npad,), -2, jnp.int32)]).reshape(1, N_PAD)

    seg = jnp.searchsorted(bi, jnp.arange(11, dtype=jnp.int32)).astype(jnp.int32)
    r0s = jnp.arange(NBR, dtype=jnp.int32) * R
    blo = br[r0s, 0]
    bhi = br[jnp.minimum(r0s + (R - 1), N_PAD - 1), 0]
    clo = seg[blo]
    chi = seg[bhi + 1]
    tlo = (clo // CT).astype(jnp.int32)
    thi = ((chi + (CT - 1)) // CT).astype(jnp.int32)

    d2n, d2p, idxsel = _knn_call(pos8, poscT, br, bc, tlo, thi)
    idxf = idxsel.reshape(E)
    d2n3 = d2n.reshape(E // EB, 1, EB)
    d2p3 = d2p.reshape(E // EB, 1, EB)

    wr_aug = jnp.zeros((3, 40, FILT), jnp.float32)
    wr_aug = wr_aug.at[:, :NRAD, :].set(W_rbf).at[:, NRAD, :].set(b_rbf)

    zp = jnp.concatenate([z.astype(jnp.int32), jnp.zeros((npad,), jnp.int32)]).reshape(N_PAD, 1)
    etP = jnp.zeros((128, 128), jnp.float32).at[:emb_table.shape[0], :emb_table.shape[1]].set(emb_table)
    weP = jnp.zeros((128, HID), jnp.float32).at[:W_emb.shape[0], :].set(W_emb)
    x, h = _emb_call(zp, etP, weP, b_emb.reshape(1, HID), W_lin1[0])

    wm = _edge_call(d2n3, d2p3, wr_aug[0])
    zero_f = jnp.zeros((1, FILT), jnp.float32)
    for blk in range(3):
        agg = _msg_call(h, wm, idxf)
        if blk < 2:
            wm = _edge_call(d2n3, d2p3, wr_aug[blk + 1])
            wn, bn = W_lin1[blk + 1], zero_f
        else:
            wn, bn = W_out, b_out.reshape(1, HID)
        x, h = _blk_call(x, agg, W_lin2[blk], b_lin2[blk].reshape(1, HID),
                         W_fc1[blk], b_fc1[blk].reshape(1, HID),
                         W_fc2[blk], b_fc2[blk].reshape(1, HID), wn, bn)

    return h[:n]


# speedup vs baseline: 1.0982x; 1.0982x over previous
import functools
from math import pi

import jax
import jax.numpy as jnp
from jax import lax
from jax.experimental import pallas as pl
from jax.experimental.pallas import tpu as pltpu
from jax.experimental.pallas import tpu_sc as plsc

N_PAD = 10240
KNN = 32
FILT = 128
HID = 256
NRAD = 32
CUTOFF = 5.0
CUT2 = CUTOFF * CUTOFF
BIG = 1e10
BIGI = 2 ** 30

R = 512
NBR = N_PAD // R
CT = 512
WW = KNN + CT
E = N_PAD * KNN

EB = 2048
RB = 1024


def _gelu(v):
    return jnp.float32(0.5) * v * (jnp.float32(1.0) + lax.erf(v * jnp.float32(0.7071067811865476)))


def _knn_body(tlo_ref, thi_ref, posr_ref, poscT_ref, br_ref, bc_ref,
              d2n_out, d2p_out, idx_out,
              wd_ref, wp_ref, wi_ref, bd_ref, bp_ref, bi_ref):
    i = pl.program_id(0)
    r0 = i * R
    bd_ref[...] = jnp.full((R, KNN), BIG, jnp.float32)
    bp_ref[...] = jnp.full((R, KNN), BIG, jnp.float32)
    bi_ref[...] = jnp.zeros((R, KNN), jnp.int32)
    posr = posr_ref[...]
    posr_b = posr.astype(jnp.bfloat16)
    sqr = posr[:, 0:1] * posr[:, 0:1] + posr[:, 1:2] * posr[:, 1:2] + posr[:, 2:3] * posr[:, 2:3]
    br = br_ref[...]
    rowid = r0 + lax.broadcasted_iota(jnp.int32, (R, CT), 0)
    lane_w = lax.broadcasted_iota(jnp.int32, (R, WW), 1)
    lane_k = lax.broadcasted_iota(jnp.int32, (R, KNN), 1)
    tlo = tlo_ref[i]
    thi = thi_ref[i]

    def tile_body(t, carry):
        c0 = t * CT
        pc = poscT_ref[:, pl.ds(c0, CT)]
        pc_b = pc.astype(jnp.bfloat16)
        d0 = posr[:, 0:1] - pc[0:1, :]
        d2p = d0 * d0
        d1 = posr[:, 1:2] - pc[1:2, :]
        d2p = d2p + d1 * d1
        dz = posr[:, 2:3] - pc[2:3, :]
        d2p = d2p + dz * dz
        sqc = pc[0:1, :] * pc[0:1, :] + pc[1:2, :] * pc[1:2, :] + pc[2:3, :] * pc[2:3, :]
        dot = jnp.dot(posr_b, pc_b, preferred_element_type=jnp.float32)
        d2n = jnp.maximum(sqr + sqc - 2.0 * dot, 0.0)
        bc = bc_ref[:, pl.ds(c0, CT)]
        colid = c0 + lax.broadcasted_iota(jnp.int32, (R, CT), 1)
        valid = (bc == br) & (colid != rowid)
        d2m = jnp.where(valid, d2n, BIG)
        rowmin = jnp.min(d2m, axis=1, keepdims=True)
        bestmax = jnp.max(bd_ref[...], axis=1, keepdims=True)

        @pl.when(jnp.any(rowmin < bestmax))
        def _():
            wd_ref[:, 0:KNN] = bd_ref[...]
            wd_ref[:, KNN:WW] = d2m
            wp_ref[:, 0:KNN] = bp_ref[...]
            wp_ref[:, KNN:WW] = d2p
            wi_ref[...] = bi_ref[...]

            def sel(s, c2):
                wd = wd_ref[...]
                m = jnp.min(wd, axis=1, keepdims=True)
                am = jnp.min(jnp.where(wd == m, lane_w, BIGI), axis=1, keepdims=True)
                hit = lane_w == am
                iv_old = jnp.sum(jnp.where(lane_k == am, wi_ref[...], 0), axis=1, keepdims=True)
                iv = jnp.where(am < KNN, iv_old, am - KNN + c0)
                pv = jnp.sum(jnp.where(hit, wp_ref[...], 0.0), axis=1, keepdims=True)
                bd_ref[...] = jnp.where(lane_k == s, m, bd_ref[...])
                bp_ref[...] = jnp.where(lane_k == s, pv, bp_ref[...])
                bi_ref[...] = jnp.where(lane_k == s, iv, bi_ref[...])
                wd_ref[...] = jnp.where(hit, BIG, wd)
                return c2

            lax.fori_loop(0, KNN, sel, 0)
        return carry

    lax.fori_loop(tlo, thi, tile_body, 0)
    d2n_out[...] = bd_ref[...]
    d2p_out[...] = bp_ref[...]
    idx_out[...] = bi_ref[...]


def _knn_call(pos8, poscT, br, bc, tlo, thi):
    return pl.pallas_call(
        _knn_body,
        grid=(NBR,),
        in_specs=[
            pl.BlockSpec(memory_space=pltpu.SMEM),
            pl.BlockSpec(memory_space=pltpu.SMEM),
            pl.BlockSpec((R, 8), lambda i: (i, 0)),
            pl.BlockSpec((8, N_PAD), lambda i: (0, 0)),
            pl.BlockSpec((R, 1), lambda i: (i, 0)),
            pl.BlockSpec((1, N_PAD), lambda i: (0, 0)),
        ],
        out_specs=[
            pl.BlockSpec((R, KNN), lambda i: (i, 0)),
            pl.BlockSpec((R, KNN), lambda i: (i, 0)),
            pl.BlockSpec((R, KNN), lambda i: (i, 0)),
        ],
        out_shape=[
            jax.ShapeDtypeStruct((N_PAD, KNN), jnp.float32),
            jax.ShapeDtypeStruct((N_PAD, KNN), jnp.float32),
            jax.ShapeDtypeStruct((N_PAD, KNN), jnp.int32),
        ],
        scratch_shapes=[
            pltpu.VMEM((R, WW), jnp.float32),
            pltpu.VMEM((R, WW), jnp.float32),
            pltpu.VMEM((R, KNN), jnp.int32),
            pltpu.VMEM((R, KNN), jnp.float32),
            pltpu.VMEM((R, KNN), jnp.float32),
            pltpu.VMEM((R, KNN), jnp.int32),
        ],
    )(tlo, thi, pos8, poscT, br, bc)


def _edge_body(d2n_ref, d2p_ref, wr_ref, o0, rbf_ref):
    mask = d2n_ref[0] <= jnp.float32(CUT2)
    dist = jnp.where(mask, jnp.sqrt(d2p_ref[0] + jnp.float32(1e-12)), jnp.float32(0.5 * CUTOFF))
    xq = dist * jnp.float32(1.0 / CUTOFF)
    x2 = xq * xq
    x4 = x2 * x2
    x5 = x4 * xq
    x6 = x4 * x2
    x7 = x6 * xq
    env = jnp.float32(1.0) / xq + jnp.float32(-28.0) * x5 + jnp.float32(48.0) * x6 + jnp.float32(-21.0) * x7
    theta = xq * jnp.float32(pi)
    s1 = jnp.sin(theta)
    c2 = jnp.float32(2.0) * jnp.cos(theta)
    cosw = jnp.float32(0.5) * (jnp.cos(dist * jnp.float32(pi / CUTOFF)) + jnp.float32(1.0))
    cm = cosw * mask.astype(jnp.float32)
    u = env * cm
    sm1 = jnp.zeros_like(s1)
    sb = s1
    for b in range(NRAD):
        rbf_ref[b:b + 1, :] = u * sb
        snext = c2 * sb - sm1
        sm1 = sb
        sb = snext
    rbf_ref[NRAD:NRAD + 1, :] = cm
    rbf_ref[NRAD + 1:40, :] = jnp.zeros((40 - NRAD - 1, EB), jnp.float32)
    basis = rbf_ref[...]
    dn = (((0,), (0,)), ((), ()))
    o0[...] = lax.dot_general(basis, wr_ref[...], dn, preferred_element_type=jnp.float32)


def _edge_call(d2nf, d2pf, wr_aug_b):
    return pl.pallas_call(
        _edge_body,
        grid=(E // EB,),
        in_specs=[
            pl.BlockSpec((1, 1, EB), lambda e: (e, 0, 0)),
            pl.BlockSpec((1, 1, EB), lambda e: (e, 0, 0)),
            pl.BlockSpec((40, FILT), lambda e: (0, 0)),
        ],
        out_specs=pl.BlockSpec((EB, FILT), lambda e: (e, 0)),
        out_shape=jax.ShapeDtypeStruct((E, FILT), jnp.float32),
        scratch_shapes=[pltpu.VMEM((40, EB), jnp.float32)],
    )(d2nf, d2pf, wr_aug_b)


def _emb_body(z_ref, et_ref, we_ref, be_ref, wl1_ref, x_out, h_out):
    z = z_ref[...]
    oh = (z == lax.broadcasted_iota(jnp.int32, (RB, 128), 1)).astype(jnp.float32)
    t = jnp.dot(oh, et_ref[...], preferred_element_type=jnp.float32)
    v = jnp.dot(t, we_ref[...], preferred_element_type=jnp.float32) + be_ref[...]
    x = _gelu(v)
    x_out[...] = x
    h_out[...] = jnp.dot(x, wl1_ref[...], preferred_element_type=jnp.float32)


def _emb_call(zp, etP, weP, be, wl1):
    return pl.pallas_call(
        _emb_body,
        grid=(N_PAD // RB,),
        in_specs=[
            pl.BlockSpec((RB, 1), lambda i: (i, 0)),
            pl.BlockSpec((128, 128), lambda i: (0, 0)),
            pl.BlockSpec((128, HID), lambda i: (0, 0)),
            pl.BlockSpec((1, HID), lambda i: (0, 0)),
            pl.BlockSpec((HID, FILT), lambda i: (0, 0)),
        ],
        out_specs=[
            pl.BlockSpec((RB, HID), lambda i: (i, 0)),
            pl.BlockSpec((RB, FILT), lambda i: (i, 0)),
        ],
        out_shape=[
            jax.ShapeDtypeStruct((N_PAD, HID), jnp.float32),
            jax.ShapeDtypeStruct((N_PAD, FILT), jnp.float32),
        ],
    )(zp, etP, weP, be, wl1)


def _blk_body(x_ref, agg_ref, wl2_ref, bl2_ref, wf1_ref, bf1_ref, wf2_ref, bf2_ref,
              wn_ref, bn_ref, x_out, n_out):
    x1 = x_ref[...] + jnp.dot(agg_ref[...], wl2_ref[...], preferred_element_type=jnp.float32) + bl2_ref[...]
    t = _gelu(jnp.dot(x1, wf1_ref[...], preferred_element_type=jnp.float32) + bf1_ref[...])
    x2 = x1 + jnp.dot(t, wf2_ref[...], preferred_element_type=jnp.float32) + bf2_ref[...]
    x_out[...] = x2
    n_out[...] = jnp.dot(x2, wn_ref[...], preferred_element_type=jnp.float32) + bn_ref[...]


def _blk_call(x, agg, wl2, bl2, wf1, bf1, wf2, bf2, wn, bn):
    nout = wn.shape[1]
    return pl.pallas_call(
        _blk_body,
        grid=(N_PAD // RB,),
        in_specs=[
            pl.BlockSpec((RB, HID), lambda i: (i, 0)),
            pl.BlockSpec((RB, FILT), lambda i: (i, 0)),
            pl.BlockSpec((FILT, HID), lambda i: (0, 0)),
            pl.BlockSpec((1, HID), lambda i: (0, 0)),
            pl.BlockSpec((HID, HID), lambda i: (0, 0)),
            pl.BlockSpec((1, HID), lambda i: (0, 0)),
            pl.BlockSpec((HID, HID), lambda i: (0, 0)),
            pl.BlockSpec((1, HID), lambda i: (0, 0)),
            pl.BlockSpec((HID, nout), lambda i: (0, 0)),
            pl.BlockSpec((1, nout), lambda i: (0, 0)),
        ],
        out_specs=[
            pl.BlockSpec((RB, HID), lambda i: (i, 0)),
            pl.BlockSpec((RB, nout), lambda i: (i, 0)),
        ],
        out_shape=[
            jax.ShapeDtypeStruct((N_PAD, HID), jnp.float32),
            jax.ShapeDtypeStruct((N_PAD, nout), jnp.float32),
        ],
    )(x, agg, wl2, bl2, wf1, bf1, wf2, bf2, wn, bn)


NW = 32
NPT = N_PAD // NW
G = 2
GI = G * KNN
NG = NPT // G


def _msg_sc_body(h_hbm, wm_hbm, src_hbm, out_hbm, idx_v, rows0, rows1, wmv0, wmv1,
                 agg_v, gs0, gs1, ws0, ws1):
    cid = lax.axis_index("c")
    sid = lax.axis_index("s")
    wid = sid * 2 + cid
    base = wid * NPT
    pltpu.sync_copy(src_hbm.at[pl.ds(base * KNN, NPT * KNN)], idx_v)

    rows_b = (rows0, rows1)
    wm_b = (wmv0, wmv1)
    gs_b = (gs0, gs1)
    ws_b = (ws0, ws1)

    def issue(g, buf):
        pltpu.async_copy(h_hbm.at[idx_v.at[pl.ds(g * GI, GI)]], rows_b[buf], gs_b[buf])
        pltpu.async_copy(wm_hbm.at[pl.ds(base * KNN + g * GI, GI)], wm_b[buf], ws_b[buf])

    issue(0, 0)

    def body(g2, carry):
        for par in (0, 1):
            g = 2 * g2 + par
            cur = par
            nxt = 1 - par

            @pl.when(g + 1 < NG)
            def _():
                pltpu.async_copy(h_hbm.at[idx_v.at[pl.ds((g + 1) * GI, GI)]], rows_b[nxt], gs_b[nxt])
                pltpu.async_copy(wm_hbm.at[pl.ds(base * KNN + (g + 1) * GI, GI)], wm_b[nxt], ws_b[nxt])

            pltpu.make_async_copy(h_hbm.at[idx_v.at[pl.ds(0, GI)]], rows_b[cur], gs_b[cur]).wait()
            pltpu.make_async_copy(wm_hbm.at[pl.ds(0, GI)], wm_b[cur], ws_b[cur]).wait()

            for gn in range(G):
                n = g * G + gn
                for c in range(FILT // 16):
                    acc = jnp.zeros((16,), jnp.float32)
                    for s in range(KNN):
                        e = gn * KNN + s
                        acc = acc + rows_b[cur][e, pl.ds(c * 16, 16)] * wm_b[cur][e, pl.ds(c * 16, 16)]
                    agg_v[n, pl.ds(c * 16, 16)] = acc
        return carry

    lax.fori_loop(0, NG // 2, body, 0)
    pltpu.sync_copy(agg_v, out_hbm.at[pl.ds(base, NPT)])


def _msg_call(h, wm, idxf):
    mesh = plsc.VectorSubcoreMesh(core_axis_name="c", subcore_axis_name="s")
    fn = functools.partial(
        pl.kernel,
        mesh=mesh,
        out_type=jax.ShapeDtypeStruct((N_PAD, FILT), jnp.float32),
        scratch_types=[
            pltpu.VMEM((NPT * KNN,), jnp.int32),
            pltpu.VMEM((GI, FILT), jnp.float32),
            pltpu.VMEM((GI, FILT), jnp.float32),
            pltpu.VMEM((GI, FILT), jnp.float32),
            pltpu.VMEM((GI, FILT), jnp.float32),
            pltpu.VMEM((NPT, FILT), jnp.float32),
            pltpu.SemaphoreType.DMA,
            pltpu.SemaphoreType.DMA,
            pltpu.SemaphoreType.DMA,
            pltpu.SemaphoreType.DMA,
        ],
    )(_msg_sc_body)
    return fn(h, wm, idxf)


def kernel(z, pos, batch, emb_table, W_emb, b_emb, freq, W_rbf, b_rbf,
           W_lin1, W_lin2, b_lin2, W_fc1, b_fc1, W_fc2, b_fc2, W_out, b_out):
    n = pos.shape[0]
    npad = N_PAD - n

    posf = pos.astype(jnp.float32)
    pos8 = jnp.concatenate(
        [posf, jnp.zeros((n, 5), jnp.float32)], axis=1)
    pos8 = jnp.concatenate([pos8, jnp.zeros((npad, 8), jnp.float32)], axis=0)
    poscT = pos8.T

    bi = batch.astype(jnp.int32)
    br = jnp.concatenate([bi, jnp.full((npad,), bi[-1], jnp.int32)]).reshape(N_PAD, 1)
    bc = jnp.concatenate([bi, jnp.full((npad,), -2, jnp.int32)]).reshape(1, N_PAD)

    seg = jnp.searchsorted(bi, jnp.arange(11, dtype=jnp.int32)).astype(jnp.int32)
    r0s = jnp.arange(NBR, dtype=jnp.int32) * R
    blo = br[r0s, 0]
    bhi = br[jnp.minimum(r0s + (R - 1), N_PAD - 1), 0]
    clo = seg[blo]
    chi = seg[bhi + 1]
    tlo = (clo // CT).astype(jnp.int32)
    thi = ((chi + (CT - 1)) // CT).astype(jnp.int32)

    d2n, d2p, idxsel = _knn_call(pos8, poscT, br, bc, tlo, thi)
    idxf = idxsel.reshape(E)
    d2n3 = d2n.reshape(E // EB, 1, EB)
    d2p3 = d2p.reshape(E // EB, 1, EB)

    wr_aug = jnp.zeros((3, 40, FILT), jnp.float32)
    wr_aug = wr_aug.at[:, :NRAD, :].set(W_rbf).at[:, NRAD, :].set(b_rbf)

    zp = jnp.concatenate([z.astype(jnp.int32), jnp.zeros((npad,), jnp.int32)]).reshape(N_PAD, 1)
    etP = jnp.zeros((128, 128), jnp.float32).at[:emb_table.shape[0], :emb_table.shape[1]].set(emb_table)
    weP = jnp.zeros((128, HID), jnp.float32).at[:W_emb.shape[0], :].set(W_emb)
    x, h = _emb_call(zp, etP, weP, b_emb.reshape(1, HID), W_lin1[0])

    wm = _edge_call(d2n3, d2p3, wr_aug[0])
    zero_f = jnp.zeros((1, FILT), jnp.float32)
    for blk in range(3):
        agg = _msg_call(h, wm, idxf)
        if blk < 2:
            wm = _edge_call(d2n3, d2p3, wr_aug[blk + 1])
            wn, bn = W_lin1[blk + 1], zero_f
        else:
            wn, bn = W_out, b_out.reshape(1, HID)
        x, h = _blk_call(x, agg, W_lin2[blk], b_lin2[blk].reshape(1, HID),
                         W_fc1[blk], b_fc1[blk].reshape(1, HID),
                         W_fc2[blk], b_fc2[blk].reshape(1, HID), wn, bn)

    return h[:n]
